# Initial kernel scaffold; baseline (speedup 1.0000x reference)
import jax
import jax.numpy as jnp
from jax.experimental import pallas as pl


def kernel(x, Wq, bq, Wk, bk, Wv, bv, Wo, bo, n1g, n1b, gW1, gb1, gW2, gb2, eW1, eb1, eW2, eb2, hW1, hb1, hW2, hb2, n2g, n2b):
    raise NotImplementedError("write your pallas kernel here")


# speedup vs baseline: 1.2418x; 1.2418x over previous
import functools

import jax
import jax.numpy as jnp
from jax import lax
from jax.experimental import pallas as pl
from jax.experimental.pallas import tpu as pltpu

B = 1
L = 2048
D = 768
H = 12
DH = D // H
DFF = 3072
E = 8
EPS = 1e-9
NEG = -1e30
HI = jax.lax.Precision.HIGHEST

INTERPRET = False

F32 = jnp.float32


def _ln(y, g, b):
    m = jnp.mean(y, axis=-1, keepdims=True)
    v = jnp.mean((y - m) ** 2, axis=-1, keepdims=True)
    return (y - m) / jnp.sqrt(v + 1e-5) * g + b


def _attn_body(xq_ref, xk_ref, wq_ref, bq_ref, wk_ref, bk_ref, wv_ref,
               bv_ref, attn_ref, o_ref, k2_ref, v2_ref):
    qb = pl.program_id(1)

    @pl.when(qb == 0)
    def _():
        xx = xk_ref[...]
        k2_ref[...] = jnp.dot(xx, wk_ref[...], preferred_element_type=F32,
                              precision=HI) + bk_ref[...]
        v2_ref[...] = jnp.dot(xx, wv_ref[...], preferred_element_type=F32,
                              precision=HI) + bv_ref[...]

    q2 = jnp.dot(xq_ref[...], wq_ref[...], preferred_element_type=F32,
                 precision=HI) + bq_ref[...]
    k2 = k2_ref[...]
    v2 = v2_ref[...]
    lane = lax.broadcasted_iota(jnp.int32, (1, 2 * DH), 1)
    for j in range(2):
        mj = (lane // DH == j).astype(F32)
        s = lax.dot_general((q2 * mj).astype(jnp.bfloat16),
                            k2.astype(jnp.bfloat16),
                            (((1,), (1,)), ((), ())),
                            preferred_element_type=F32) * (1.0 / 8.0)
        m = jnp.max(s, axis=-1, keepdims=True)
        p = jnp.exp(s - m)
        ssum = jnp.sum(p, axis=-1, keepdims=True)
        attn_ref[j] = p / ssum
        o_ref[:, j * DH:(j + 1) * DH] = jnp.dot(
            p.astype(jnp.bfloat16),
            v2[:, j * DH:(j + 1) * DH].astype(jnp.bfloat16),
            preferred_element_type=F32) / ssum


def _postattn_body(x_ref, o_ref, wo_ref, bo_ref, g_ref, b_ref, x1_ref):
    y = x_ref[...] + jnp.dot(o_ref[...].astype(jnp.bfloat16),
                             wo_ref[...].astype(jnp.bfloat16),
                             preferred_element_type=F32) + bo_ref[...]
    x1_ref[...] = _ln(y, g_ref[...], b_ref[...])


def _router_body(x1_ref, gw1_ref, gb1_ref, gw2_ref, gb2_ref, gated_ref):
    hg = jnp.maximum(
        jnp.dot(x1_ref[...], gw1_ref[...], preferred_element_type=F32)
        + gb1_ref[...], 0.0)
    logits = jnp.dot(hg, gw2_ref[...], preferred_element_type=F32) + gb2_ref[...]
    idx = lax.broadcasted_iota(jnp.int32, (L, E), 1)
    m1 = jnp.max(logits, axis=-1, keepdims=True)
    i1 = jnp.min(jnp.where(logits == m1, idx, E), axis=-1, keepdims=True)
    sel1 = idx == i1
    masked = jnp.where(sel1, NEG, logits)
    m2 = jnp.max(masked, axis=-1, keepdims=True)
    i2 = jnp.min(jnp.where(masked == m2, idx, E), axis=-1, keepdims=True)
    mask = sel1 | (idx == i2)
    p = jnp.exp(logits - m1)
    probs = p / jnp.sum(p, axis=-1, keepdims=True)
    g = jnp.where(mask, probs, 0.0)
    gated_ref[...] = g / (jnp.sum(g, axis=-1, keepdims=True) + EPS)


def _experts_body(nd, x1_ref, w1_ref, b1_ref, w2_ref, b2_ref, gated_ref,
                  tot_ref, acc_ref):
    e = pl.program_id(0)
    d = pl.program_id(1)
    h1 = jnp.maximum(
        jnp.dot(x1_ref[...], w1_ref[0], preferred_element_type=F32)
        + b1_ref[0], 0.0)
    part = jnp.dot(h1, w2_ref[0], preferred_element_type=F32)

    @pl.when(d == 0)
    def _():
        acc_ref[...] = part

    @pl.when(d != 0)
    def _():
        acc_ref[...] += part

    @pl.when(d == nd - 1)
    def _():
        eidx = lax.broadcasted_iota(jnp.int32, (L, E), 1)
        w = jnp.sum(jnp.where(eidx == e, gated_ref[...], 0.0), axis=-1,
                    keepdims=True)
        contrib = (acc_ref[...] + b2_ref[0]) * w

        @pl.when(e == 0)
        def _():
            tot_ref[...] = contrib

        @pl.when(e != 0)
        def _():
            tot_ref[...] += contrib


def _final_body(nd, x1_ref, w1_ref, b1_ref, w2_ref, b2_ref, tot_ref,
                g_ref, b_ref, out_ref, acc_ref):
    d = pl.program_id(0)
    h1 = jnp.maximum(
        jnp.dot(x1_ref[...], w1_ref[...], preferred_element_type=F32)
        + b1_ref[...], 0.0)
    part = jnp.dot(h1, w2_ref[...], preferred_element_type=F32)

    @pl.when(d == 0)
    def _():
        acc_ref[...] = part

    @pl.when(d != 0)
    def _():
        acc_ref[...] += part

    @pl.when(d == nd - 1)
    def _():
        gen = acc_ref[...] + b2_ref[...]
        t32 = tot_ref[...].astype(jnp.bfloat16).astype(F32)
        y = gen + t32 + x1_ref[...]
        out_ref[...] = _ln(y, g_ref[...], b_ref[...])


def kernel(x, Wq, bq, Wk, bk, Wv, bv, Wo, bo, n1g, n1b, gW1, gb1, gW2, gb2,
           eW1, eb1, eW2, eb2, hW1, hb1, hW2, hb2, n2g, n2b):
    xf = x.reshape(L, D)
    r2 = lambda v: v.reshape(1, -1)

    QBLK = 512
    NQB = L // QBLK
    DH2 = 2 * DH
    attn, o = pl.pallas_call(
        _attn_body,
        grid=(H // 2, NQB),
        in_specs=[
            pl.BlockSpec((QBLK, D), lambda hp, qb: (qb, 0)),
            pl.BlockSpec((L, D), lambda hp, qb: (0, 0)),
            pl.BlockSpec((D, DH2), lambda hp, qb: (0, hp)),
            pl.BlockSpec((1, DH2), lambda hp, qb: (0, hp)),
            pl.BlockSpec((D, DH2), lambda hp, qb: (0, hp)),
            pl.BlockSpec((1, DH2), lambda hp, qb: (0, hp)),
            pl.BlockSpec((D, DH2), lambda hp, qb: (0, hp)),
            pl.BlockSpec((1, DH2), lambda hp, qb: (0, hp)),
        ],
        out_specs=[
            pl.BlockSpec((2, QBLK, L), lambda hp, qb: (hp, qb, 0)),
            pl.BlockSpec((QBLK, DH2), lambda hp, qb: (qb, hp)),
        ],
        out_shape=[
            jax.ShapeDtypeStruct((H, L, L), F32),
            jax.ShapeDtypeStruct((L, D), F32),
        ],
        scratch_shapes=[
            pltpu.VMEM((L, DH2), F32),
            pltpu.VMEM((L, DH2), F32),
        ],
        interpret=INTERPRET,
    )(xf, xf, Wq, r2(bq), Wk, r2(bk), Wv, r2(bv))

    x1 = pl.pallas_call(
        _postattn_body,
        out_shape=jax.ShapeDtypeStruct((L, D), F32),
        interpret=INTERPRET,
    )(xf, o, Wo, r2(bo), r2(n1g), r2(n1b))

    gated = pl.pallas_call(
        _router_body,
        out_shape=jax.ShapeDtypeStruct((L, E), F32),
        interpret=INTERPRET,
    )(x1, gW1, r2(gb1), gW2, r2(gb2))

    ND = 2
    FD = DFF // ND
    total = pl.pallas_call(
        functools.partial(_experts_body, ND),
        grid=(E, ND),
        in_specs=[
            pl.BlockSpec((L, D), lambda e, d: (0, 0)),
            pl.BlockSpec((1, D, FD), lambda e, d: (e, 0, d)),
            pl.BlockSpec((1, 1, FD), lambda e, d: (e, 0, d)),
            pl.BlockSpec((1, FD, D), lambda e, d: (e, d, 0)),
            pl.BlockSpec((1, 1, D), lambda e, d: (e, 0, 0)),
            pl.BlockSpec((L, E), lambda e, d: (0, 0)),
        ],
        out_specs=pl.BlockSpec((L, D), lambda e, d: (0, 0)),
        out_shape=jax.ShapeDtypeStruct((L, D), F32),
        scratch_shapes=[pltpu.VMEM((L, D), F32)],
        interpret=INTERPRET,
    )(x1, eW1, eb1.reshape(E, 1, DFF), eW2, eb2.reshape(E, 1, D), gated)

    ND5 = 4
    FD5 = DFF // ND5
    out = pl.pallas_call(
        functools.partial(_final_body, ND5),
        grid=(ND5,),
        in_specs=[
            pl.BlockSpec((L, D), lambda d: (0, 0)),
            pl.BlockSpec((D, FD5), lambda d: (0, d)),
            pl.BlockSpec((1, FD5), lambda d: (0, d)),
            pl.BlockSpec((FD5, D), lambda d: (d, 0)),
            pl.BlockSpec((1, D), lambda d: (0, 0)),
            pl.BlockSpec((L, D), lambda d: (0, 0)),
            pl.BlockSpec((1, D), lambda d: (0, 0)),
            pl.BlockSpec((1, D), lambda d: (0, 0)),
        ],
        out_specs=pl.BlockSpec((L, D), lambda d: (0, 0)),
        out_shape=jax.ShapeDtypeStruct((L, D), F32),
        scratch_shapes=[pltpu.VMEM((L, D), F32)],
        interpret=INTERPRET,
    )(x1, hW1, r2(hb1), hW2, r2(hb2), total, r2(n2g), r2(n2b))

    return (out.reshape(B, L, D), attn.reshape(B, H, L, L), jnp.float32(0.0))
